# baseline (speedup 1.0000x reference)
import functools

import jax
import jax.numpy as jnp
from jax.experimental import pallas as pl
from jax.experimental.pallas import tpu as pltpu


def _select_body(tgt_ref, mask_ref, ids_ref, pos_ref, out_ref, *, A):
    ids = ids_ref[...]
    t0, t1, t2 = tgt_ref[0], tgt_ref[1], tgt_ref[2]
    tmask = (ids == t0) | (ids == t1) | (ids == t2)
    sel = tmask & mask_ref[0]
    aidx = jax.lax.broadcasted_iota(jnp.int32, sel.shape, 1)
    big = jnp.int32(A + 1)
    first = jnp.min(jnp.where(sel, aidx, big), axis=1, keepdims=True)

    pos = pos_ref[0]
    cidx = jax.lax.broadcasted_iota(jnp.int32, pos.shape, 1) // 3
    keep = (cidx == first).astype(jnp.float32)
    masked = pos * keep

    rows = jax.lax.broadcasted_iota(jnp.int32, (3 * A, 4), 0) % 3
    cols = jax.lax.broadcasted_iota(jnp.int32, (3 * A, 4), 1)
    p = (rows == cols).astype(jnp.float32)
    q = ((rows == 0) & (cols == 3)).astype(jnp.float32)
    out = jnp.dot(masked, p, preferred_element_type=jnp.float32,
                  precision=jax.lax.Precision.HIGHEST)
    out += jnp.dot(keep, q, preferred_element_type=jnp.float32,
                   precision=jax.lax.Precision.HIGHEST)
    out_ref[0] = out


def kernel(pos_atoms, mask_atoms, atom_name_ids, target_ids):
    N, L, A, _ = pos_atoms.shape
    pos3a = pos_atoms.reshape(N, L, 3 * A)
    Lb = min(1024, L)
    grid = (L // Lb, N)

    out4 = pl.pallas_call(
        functools.partial(_select_body, A=A),
        grid=grid,
        in_specs=[
            pl.BlockSpec(memory_space=pltpu.SMEM),
            pl.BlockSpec((1, Lb, A), lambda jl, n: (n, jl, 0)),
            pl.BlockSpec((Lb, A), lambda jl, n: (jl, 0)),
            pl.BlockSpec((1, Lb, 3 * A), lambda jl, n: (n, jl, 0)),
        ],
        out_specs=pl.BlockSpec((1, Lb, 4), lambda jl, n: (n, jl, 0)),
        out_shape=jax.ShapeDtypeStruct((N, L, 4), jnp.float32),
        compiler_params=pltpu.CompilerParams(
            dimension_semantics=("parallel", "arbitrary"),
        ),
    )(target_ids, mask_atoms, atom_name_ids, pos3a)

    return out4[:, :, :3], out4[:, :, 3]


# speedup vs baseline: 1.1616x; 1.1616x over previous
import functools

import jax
import jax.numpy as jnp
from jax.experimental import pallas as pl
from jax.experimental.pallas import tpu as pltpu


def _select_body(tgt_ref, mask_ref, ids_ref, pos_ref, out_ref, *, A):
    ids = ids_ref[...]
    t0, t1, t2 = tgt_ref[0], tgt_ref[1], tgt_ref[2]
    tmask = (ids == t0) | (ids == t1) | (ids == t2)
    sel = tmask & mask_ref[0]
    aidx = jax.lax.broadcasted_iota(jnp.int32, sel.shape, 1)
    big = jnp.int32(A + 1)
    first = jnp.min(jnp.where(sel, aidx, big), axis=1, keepdims=True)

    pos = pos_ref[0]
    cidx = jax.lax.broadcasted_iota(jnp.int32, pos.shape, 1) // 3
    keep = (cidx == first).astype(jnp.float32)
    masked = pos * keep

    rows = jax.lax.broadcasted_iota(jnp.int32, (3 * A, 4), 0) % 3
    cols = jax.lax.broadcasted_iota(jnp.int32, (3 * A, 4), 1)
    p = (rows == cols).astype(jnp.float32)
    q = ((rows == 0) & (cols == 3)).astype(jnp.float32)
    out = jnp.dot(masked, p, preferred_element_type=jnp.float32)
    out += jnp.dot(keep, q, preferred_element_type=jnp.float32)
    out_ref[0] = out


def kernel(pos_atoms, mask_atoms, atom_name_ids, target_ids):
    N, L, A, _ = pos_atoms.shape
    pos3a = pos_atoms.reshape(N, L, 3 * A)
    Lb = min(1024, L)
    grid = (L // Lb, N)

    out4 = pl.pallas_call(
        functools.partial(_select_body, A=A),
        grid=grid,
        in_specs=[
            pl.BlockSpec(memory_space=pltpu.SMEM),
            pl.BlockSpec((1, Lb, A), lambda jl, n: (n, jl, 0)),
            pl.BlockSpec((Lb, A), lambda jl, n: (jl, 0)),
            pl.BlockSpec((1, Lb, 3 * A), lambda jl, n: (n, jl, 0)),
        ],
        out_specs=pl.BlockSpec((1, Lb, 4), lambda jl, n: (n, jl, 0)),
        out_shape=jax.ShapeDtypeStruct((N, L, 4), jnp.float32),
        compiler_params=pltpu.CompilerParams(
            dimension_semantics=("parallel", "arbitrary"),
        ),
    )(target_ids, mask_atoms, atom_name_ids, pos3a)

    return out4[:, :, :3], out4[:, :, 3]
